# baseline (speedup 1.0000x reference)
import functools

import jax
import jax.numpy as jnp
from jax import lax
from jax.experimental import pallas as pl
from jax.experimental.pallas import tpu as pltpu
from jax.experimental.pallas import tpu_sc as plsc

N_TOKENS = 204800
N_BAGS = 4096
VOCAB = 100000
EMB_DIM = 64

NC = 2
NS = 16
NW = NC * NS
NBW = N_BAGS // NW
CHUNK = 512
NIDX = 128
LANES = 16
DV = EMB_DIM // LANES


def _splat(val):
    return jnp.full((LANES,), val, jnp.int32)


def _body(inp_hbm, offs_hbm, w_hbm, tab_hbm, out_hbm,
          offs_v, offs2_v, idx0_v, idx1_v, idx2_v, w0_v, w1_v, w2_v,
          rows0_v, rows1_v, rows2_v, acc_v, sem0, sem1, sem2, isem):
    cid = lax.axis_index("c")
    sid = lax.axis_index("s")
    wid = sid * NC + cid
    b0 = wid * NBW

    pltpu.sync_copy(offs_hbm.at[pl.ds(b0, NBW)], offs_v)
    nxt = jnp.minimum(b0 + NBW, N_BAGS - LANES)
    pltpu.sync_copy(offs_hbm.at[pl.ds(nxt, LANES)], offs2_v)

    t0 = offs_v[pl.ds(0, LANES)][0]
    t1 = jnp.where(wid == NW - 1, N_TOKENS, offs2_v[pl.ds(0, LANES)][0])

    zero16f = jnp.zeros((LANES,), jnp.float32)

    def zbody(i, _):
        for k in range(DV):
            acc_v[i, pl.ds(k * LANES, LANES)] = zero16f
        return 0

    lax.fori_loop(0, NBW, zbody, 0)

    sems = (sem0, sem1, sem2)
    idxs = (idx0_v, idx1_v, idx2_v)
    ws = (w0_v, w1_v, w2_v)
    rows = (rows0_v, rows1_v, rows2_v)

    def _cs(c):
        return jnp.minimum(c * CHUNK, N_TOKENS - CHUNK)

    def fire_idx(c, buf):
        s = _cs(c)
        for j in range(CHUNK // NIDX):
            pltpu.async_copy(inp_hbm.at[pl.ds(s + j * NIDX, NIDX)],
                             idxs[buf].at[j], isem)

    def wait_idx(buf):
        for j in range(CHUNK // NIDX):
            pltpu.make_async_copy(inp_hbm.at[pl.ds(0, NIDX)],
                                  idxs[buf].at[j], isem).wait()

    def fire_gathers(c, buf):
        s = _cs(c)
        pltpu.async_copy(w_hbm.at[pl.ds(s, CHUNK)],
                         ws[buf].at[pl.ds(0, CHUNK)], sems[buf])
        for j in range(CHUNK // NIDX):
            pltpu.async_copy(
                tab_hbm.at[idxs[buf].at[j]],
                rows[buf].at[pl.ds(j * NIDX, NIDX)], sems[buf])

    def gather_waits(buf):
        waits = [pltpu.make_async_copy(w_hbm.at[pl.ds(0, CHUNK)],
                                       ws[buf].at[pl.ds(0, CHUNK)],
                                       sems[buf])]
        for j in range(CHUNK // NIDX):
            waits.append(pltpu.make_async_copy(
                tab_hbm.at[idxs[buf].at[j]],
                rows[buf].at[pl.ds(j * NIDX, NIDX)], sems[buf]))
        return waits

    def process(c, buf, carry):
        p, cur, a0, a1, a2, a3 = carry
        s = c * CHUNK
        hi = jnp.minimum(t1, s + CHUNK)

        S = jnp.int32(0)
        for step in (64, 32, 16, 8, 4, 2, 1, 1):
            idx = S + step
            probe = jnp.minimum(idx - 1, NBW - 1)
            val = plsc.load_gather(offs_v, [_splat(probe)])[0]
            S = jnp.where(jnp.logical_and(idx <= NBW, val <= hi), idx, S)

        def tok_loop(lo, hi_, a0, a1, a2, a3):
            rv = rows[buf]

            def tok(k, st):
                a0, a1, a2, a3, c0, c1, c2, c3 = st
                i = lo + 4 * k
                li = i - s
                li1 = jnp.minimum(li + 1, CHUNK - 1)
                li2 = jnp.minimum(li + 2, CHUNK - 1)
                li3 = jnp.minimum(li + 3, CHUNK - 1)
                wv = ws[buf][pl.ds(li, LANES)]
                wa = jnp.full((LANES,), wv[0], jnp.float32)
                wb = jnp.full((LANES,), wv[1], jnp.float32)
                wc = jnp.full((LANES,), wv[2], jnp.float32)
                wd = jnp.full((LANES,), wv[3], jnp.float32)
                wb = jnp.where(i + 1 < hi_, wb, zero16f)
                wc = jnp.where(i + 2 < hi_, wc, zero16f)
                wd = jnp.where(i + 3 < hi_, wd, zero16f)
                a0 = a0 + wa * rv[li, pl.ds(0, LANES)]
                c0 = c0 + wb * rv[li1, pl.ds(0, LANES)]
                a1 = a1 + wa * rv[li, pl.ds(LANES, LANES)]
                c1 = c1 + wb * rv[li1, pl.ds(LANES, LANES)]
                a2 = a2 + wa * rv[li, pl.ds(2 * LANES, LANES)]
                c2 = c2 + wb * rv[li1, pl.ds(2 * LANES, LANES)]
                a3 = a3 + wa * rv[li, pl.ds(3 * LANES, LANES)]
                c3 = c3 + wb * rv[li1, pl.ds(3 * LANES, LANES)]
                a0 = a0 + wc * rv[li2, pl.ds(0, LANES)]
                c0 = c0 + wd * rv[li3, pl.ds(0, LANES)]
                a1 = a1 + wc * rv[li2, pl.ds(LANES, LANES)]
                c1 = c1 + wd * rv[li3, pl.ds(LANES, LANES)]
                a2 = a2 + wc * rv[li2, pl.ds(2 * LANES, LANES)]
                c2 = c2 + wd * rv[li3, pl.ds(2 * LANES, LANES)]
                a3 = a3 + wc * rv[li2, pl.ds(3 * LANES, LANES)]
                c3 = c3 + wd * rv[li3, pl.ds(3 * LANES, LANES)]
                return (a0, a1, a2, a3, c0, c1, c2, c3)

            n4 = (hi_ - lo + 3) // 4
            a0, a1, a2, a3, c0, c1, c2, c3 = lax.fori_loop(
                0, n4, tok,
                (a0, a1, a2, a3, zero16f, zero16f, zero16f, zero16f))
            return (a0 + c0, a1 + c1, a2 + c2, a3 + c3)

        def bag_body(k, st):
            p, a0, a1, a2, a3 = st
            nb = plsc.load_gather(offs_v, [_splat(k + 1)])[0]
            a0, a1, a2, a3 = tok_loop(p, nb, a0, a1, a2, a3)
            acc_v[k, pl.ds(0, LANES)] = a0
            acc_v[k, pl.ds(LANES, LANES)] = a1
            acc_v[k, pl.ds(2 * LANES, LANES)] = a2
            acc_v[k, pl.ds(3 * LANES, LANES)] = a3
            return (nb, zero16f, zero16f, zero16f, zero16f)

        p, a0, a1, a2, a3 = lax.fori_loop(cur, S - 1, bag_body,
                                          (p, a0, a1, a2, a3))
        cur = jnp.maximum(cur, S - 1)

        a0, a1, a2, a3 = tok_loop(p, hi, a0, a1, a2, a3)
        return (hi, cur, a0, a1, a2, a3)

    c_start = t0 // CHUNK
    c_end = (t1 + CHUNK - 1) // CHUNK
    ntriples = (c_end - c_start + 2) // 3

    for j in range(CHUNK // NIDX):
        pltpu.sync_copy(inp_hbm.at[pl.ds(_cs(c_start) + j * NIDX, NIDX)],
                        idxs[0].at[j])
        pltpu.sync_copy(inp_hbm.at[pl.ds(_cs(c_start + 1) + j * NIDX, NIDX)],
                        idxs[1].at[j])
    fire_gathers(c_start, 0)
    fire_gathers(c_start + 1, 1)
    fire_idx(c_start + 2, 2)

    def triple_body(k, carry):
        c0 = c_start + 3 * k
        for j in range(3):
            c = c0 + j
            b = j
            bn = (j + 2) % 3
            wait_idx(bn)
            fire_gathers(c + 2, bn)
            for cp in gather_waits(b):
                cp.wait()
            fire_idx(c + 3, b)
            carry = process(c, b, carry)
        return carry

    init = (t0, jnp.int32(0), zero16f, zero16f, zero16f, zero16f)
    p, cur, a0, a1, a2, a3 = lax.fori_loop(0, ntriples, triple_body, init)

    for cp in gather_waits(0):
        cp.wait()
    for cp in gather_waits(1):
        cp.wait()
    wait_idx(2)

    ci = jnp.minimum(cur, NBW)
    acc_v[ci, pl.ds(0, LANES)] = a0
    acc_v[ci, pl.ds(LANES, LANES)] = a1
    acc_v[ci, pl.ds(2 * LANES, LANES)] = a2
    acc_v[ci, pl.ds(3 * LANES, LANES)] = a3

    pltpu.sync_copy(acc_v.at[pl.ds(0, NBW)], out_hbm.at[pl.ds(b0, NBW)])


@functools.cache
def _build():
    mesh = plsc.VectorSubcoreMesh(core_axis_name="c", subcore_axis_name="s")
    return pl.kernel(
        _body,
        out_type=jax.ShapeDtypeStruct((N_BAGS, EMB_DIM), jnp.float32),
        mesh=mesh,
        scratch_types=[
            pltpu.VMEM((NBW,), jnp.int32),
            pltpu.VMEM((LANES,), jnp.int32),
            pltpu.VMEM((CHUNK // NIDX, NIDX), jnp.int32),
            pltpu.VMEM((CHUNK // NIDX, NIDX), jnp.int32),
            pltpu.VMEM((CHUNK // NIDX, NIDX), jnp.int32),
            pltpu.VMEM((CHUNK + LANES,), jnp.float32),
            pltpu.VMEM((CHUNK + LANES,), jnp.float32),
            pltpu.VMEM((CHUNK + LANES,), jnp.float32),
            pltpu.VMEM((CHUNK, EMB_DIM), jnp.float32),
            pltpu.VMEM((CHUNK, EMB_DIM), jnp.float32),
            pltpu.VMEM((CHUNK, EMB_DIM), jnp.float32),
            pltpu.VMEM((NBW + 1, EMB_DIM), jnp.float32),
            pltpu.SemaphoreType.DMA,
            pltpu.SemaphoreType.DMA,
            pltpu.SemaphoreType.DMA,
            pltpu.SemaphoreType.DMA,
        ],
        compiler_params=pltpu.CompilerParams(needs_layout_passes=False,
                                             use_tc_tiling_on_sc=False),
        name="emb_bag_segment_sum",
    )


@jax.jit
def kernel(input_, offsets, emb_weights, emb_table):
    fn = _build()
    return fn(input_.astype(jnp.int32), offsets.astype(jnp.int32),
              emb_weights, emb_table)


# speedup vs baseline: 1.0405x; 1.0405x over previous
import functools

import jax
import jax.numpy as jnp
from jax import lax
from jax.experimental import pallas as pl
from jax.experimental.pallas import tpu as pltpu
from jax.experimental.pallas import tpu_sc as plsc

N_TOKENS = 204800
N_BAGS = 4096
VOCAB = 100000
EMB_DIM = 64

NC = 2
NS = 16
NW = NC * NS
NBW = N_BAGS // NW
CHUNK = 256
NIDX = 128
LANES = 16
DV = EMB_DIM // LANES


def _splat(val):
    return jnp.full((LANES,), val, jnp.int32)


def _body(inp_hbm, offs_hbm, w_hbm, tab_hbm, out_hbm,
          offs_v, offs2_v, idx0_v, idx1_v, idx2_v, w0_v, w1_v, w2_v,
          rows0_v, rows1_v, rows2_v, acc_v, sem0, sem1, sem2, isem):
    cid = lax.axis_index("c")
    sid = lax.axis_index("s")
    wid = sid * NC + cid
    b0 = wid * NBW

    pltpu.sync_copy(offs_hbm.at[pl.ds(b0, NBW)], offs_v)
    nxt = jnp.minimum(b0 + NBW, N_BAGS - LANES)
    pltpu.sync_copy(offs_hbm.at[pl.ds(nxt, LANES)], offs2_v)

    t0 = offs_v[pl.ds(0, LANES)][0]
    t1 = jnp.where(wid == NW - 1, N_TOKENS, offs2_v[pl.ds(0, LANES)][0])

    zero16f = jnp.zeros((LANES,), jnp.float32)

    def zbody(i, _):
        for k in range(DV):
            acc_v[i, pl.ds(k * LANES, LANES)] = zero16f
        return 0

    lax.fori_loop(0, NBW, zbody, 0)

    sems = (sem0, sem1, sem2)
    idxs = (idx0_v, idx1_v, idx2_v)
    ws = (w0_v, w1_v, w2_v)
    rows = (rows0_v, rows1_v, rows2_v)

    def _cs(c):
        return jnp.minimum(c * CHUNK, N_TOKENS - CHUNK)

    def fire_idx(c, buf):
        pltpu.async_copy(inp_hbm.at[pl.ds(_cs(c), CHUNK)], idxs[buf], isem)

    def wait_idx(buf):
        pltpu.make_async_copy(inp_hbm.at[pl.ds(0, CHUNK)], idxs[buf],
                              isem).wait()

    def fire_gathers(c, buf):
        s = _cs(c)
        pltpu.async_copy(w_hbm.at[pl.ds(s, CHUNK)],
                         ws[buf].at[pl.ds(0, CHUNK)], sems[buf])
        for j in range(CHUNK // NIDX):
            pltpu.async_copy(
                tab_hbm.at[idxs[buf].at[pl.ds(j * NIDX, NIDX)]],
                rows[buf].at[pl.ds(j * NIDX, NIDX)], sems[buf])

    def gather_waits(buf):
        waits = [pltpu.make_async_copy(w_hbm.at[pl.ds(0, CHUNK)],
                                       ws[buf].at[pl.ds(0, CHUNK)],
                                       sems[buf])]
        for j in range(CHUNK // NIDX):
            waits.append(pltpu.make_async_copy(
                tab_hbm.at[idxs[buf].at[pl.ds(j * NIDX, NIDX)]],
                rows[buf].at[pl.ds(j * NIDX, NIDX)], sems[buf]))
        return waits

    def process(c, buf, carry):
        p, cur, a0, a1, a2, a3 = carry
        s = c * CHUNK
        hi = jnp.minimum(t1, s + CHUNK)

        S = jnp.int32(0)
        for step in (64, 32, 16, 8, 4, 2, 1, 1):
            idx = S + step
            probe = jnp.minimum(idx - 1, NBW - 1)
            val = plsc.load_gather(offs_v, [_splat(probe)])[0]
            S = jnp.where(jnp.logical_and(idx <= NBW, val <= hi), idx, S)

        def tok_loop(lo, hi_, a0, a1, a2, a3):
            rv = rows[buf]

            def tok(k, st):
                a0, a1, a2, a3, c0, c1, c2, c3 = st
                i = lo + 4 * k
                li = i - s
                li1 = jnp.minimum(li + 1, CHUNK - 1)
                li2 = jnp.minimum(li + 2, CHUNK - 1)
                li3 = jnp.minimum(li + 3, CHUNK - 1)
                wv = ws[buf][pl.ds(li, LANES)]
                wa = jnp.full((LANES,), wv[0], jnp.float32)
                wb = jnp.full((LANES,), wv[1], jnp.float32)
                wc = jnp.full((LANES,), wv[2], jnp.float32)
                wd = jnp.full((LANES,), wv[3], jnp.float32)
                wb = jnp.where(i + 1 < hi_, wb, zero16f)
                wc = jnp.where(i + 2 < hi_, wc, zero16f)
                wd = jnp.where(i + 3 < hi_, wd, zero16f)
                a0 = a0 + wa * rv[li, pl.ds(0, LANES)]
                c0 = c0 + wb * rv[li1, pl.ds(0, LANES)]
                a1 = a1 + wa * rv[li, pl.ds(LANES, LANES)]
                c1 = c1 + wb * rv[li1, pl.ds(LANES, LANES)]
                a2 = a2 + wa * rv[li, pl.ds(2 * LANES, LANES)]
                c2 = c2 + wb * rv[li1, pl.ds(2 * LANES, LANES)]
                a3 = a3 + wa * rv[li, pl.ds(3 * LANES, LANES)]
                c3 = c3 + wb * rv[li1, pl.ds(3 * LANES, LANES)]
                a0 = a0 + wc * rv[li2, pl.ds(0, LANES)]
                c0 = c0 + wd * rv[li3, pl.ds(0, LANES)]
                a1 = a1 + wc * rv[li2, pl.ds(LANES, LANES)]
                c1 = c1 + wd * rv[li3, pl.ds(LANES, LANES)]
                a2 = a2 + wc * rv[li2, pl.ds(2 * LANES, LANES)]
                c2 = c2 + wd * rv[li3, pl.ds(2 * LANES, LANES)]
                a3 = a3 + wc * rv[li2, pl.ds(3 * LANES, LANES)]
                c3 = c3 + wd * rv[li3, pl.ds(3 * LANES, LANES)]
                return (a0, a1, a2, a3, c0, c1, c2, c3)

            n4 = (hi_ - lo + 3) // 4
            a0, a1, a2, a3, c0, c1, c2, c3 = lax.fori_loop(
                0, n4, tok,
                (a0, a1, a2, a3, zero16f, zero16f, zero16f, zero16f))
            return (a0 + c0, a1 + c1, a2 + c2, a3 + c3)

        def bag_body(k, st):
            p, a0, a1, a2, a3 = st
            nb = plsc.load_gather(offs_v, [_splat(k + 1)])[0]
            a0, a1, a2, a3 = tok_loop(p, nb, a0, a1, a2, a3)
            acc_v[k, pl.ds(0, LANES)] = a0
            acc_v[k, pl.ds(LANES, LANES)] = a1
            acc_v[k, pl.ds(2 * LANES, LANES)] = a2
            acc_v[k, pl.ds(3 * LANES, LANES)] = a3
            return (nb, zero16f, zero16f, zero16f, zero16f)

        p, a0, a1, a2, a3 = lax.fori_loop(cur, S - 1, bag_body,
                                          (p, a0, a1, a2, a3))
        cur = jnp.maximum(cur, S - 1)

        a0, a1, a2, a3 = tok_loop(p, hi, a0, a1, a2, a3)
        return (hi, cur, a0, a1, a2, a3)

    c_start = t0 // CHUNK
    c_end = (t1 + CHUNK - 1) // CHUNK
    ntriples = (c_end - c_start + 2) // 3

    pltpu.sync_copy(inp_hbm.at[pl.ds(_cs(c_start), CHUNK)], idxs[0])
    fire_gathers(c_start, 0)
    pltpu.sync_copy(inp_hbm.at[pl.ds(_cs(c_start + 1), CHUNK)], idxs[1])
    fire_gathers(c_start + 1, 1)
    fire_idx(c_start + 2, 2)

    def triple_body(k, carry):
        c0 = c_start + 3 * k
        for j in range(3):
            c = c0 + j
            b = j
            bn = (j + 2) % 3
            wait_idx(bn)
            fire_gathers(c + 2, bn)
            for cp in gather_waits(b):
                cp.wait()
            fire_idx(c + 3, b)
            carry = process(c, b, carry)
        return carry

    init = (t0, jnp.int32(0), zero16f, zero16f, zero16f, zero16f)
    p, cur, a0, a1, a2, a3 = lax.fori_loop(0, ntriples, triple_body, init)

    for cp in gather_waits(0):
        cp.wait()
    for cp in gather_waits(1):
        cp.wait()
    wait_idx(2)

    ci = jnp.minimum(cur, NBW)
    acc_v[ci, pl.ds(0, LANES)] = a0
    acc_v[ci, pl.ds(LANES, LANES)] = a1
    acc_v[ci, pl.ds(2 * LANES, LANES)] = a2
    acc_v[ci, pl.ds(3 * LANES, LANES)] = a3

    pltpu.sync_copy(acc_v.at[pl.ds(0, NBW)], out_hbm.at[pl.ds(b0, NBW)])


@functools.cache
def _build():
    mesh = plsc.VectorSubcoreMesh(core_axis_name="c", subcore_axis_name="s")
    return pl.kernel(
        _body,
        out_type=jax.ShapeDtypeStruct((N_BAGS, EMB_DIM), jnp.float32),
        mesh=mesh,
        scratch_types=[
            pltpu.VMEM((NBW,), jnp.int32),
            pltpu.VMEM((LANES,), jnp.int32),
            pltpu.VMEM((CHUNK,), jnp.int32),
            pltpu.VMEM((CHUNK,), jnp.int32),
            pltpu.VMEM((CHUNK,), jnp.int32),
            pltpu.VMEM((CHUNK + LANES,), jnp.float32),
            pltpu.VMEM((CHUNK + LANES,), jnp.float32),
            pltpu.VMEM((CHUNK + LANES,), jnp.float32),
            pltpu.VMEM((CHUNK, EMB_DIM), jnp.float32),
            pltpu.VMEM((CHUNK, EMB_DIM), jnp.float32),
            pltpu.VMEM((CHUNK, EMB_DIM), jnp.float32),
            pltpu.VMEM((NBW + 1, EMB_DIM), jnp.float32),
            pltpu.SemaphoreType.DMA,
            pltpu.SemaphoreType.DMA,
            pltpu.SemaphoreType.DMA,
            pltpu.SemaphoreType.DMA,
        ],
        compiler_params=pltpu.CompilerParams(needs_layout_passes=False,
                                             use_tc_tiling_on_sc=False),
        name="emb_bag_segment_sum",
    )


@jax.jit
def kernel(input_, offsets, emb_weights, emb_table):
    fn = _build()
    return fn(input_.astype(jnp.int32), offsets.astype(jnp.int32),
              emb_weights, emb_table)
